# baseline (speedup 1.0000x reference)
import jax
import jax.numpy as jnp
from jax import lax
from jax.experimental import pallas as pl
from jax.experimental.pallas import tpu as pltpu
from jax.experimental.pallas import tpu_sc as plsc

N = 100000
M = 100000
K = 128
B = 16384
NC = 2
NS = 16
NW = NC * NS
BW = B // NW
BC = 64
NSUB = BW // BC
G = BC // 16
KG = K // 16
FS = 24


def _sc_body(rows_hbm, cols_hbm, l_hbm, rt_hbm, out_hbm,
             rows_v, cols_v, lrows0, rrows0, lrows1, rrows1,
             fold_v, out_v, sem_l0, sem_r0, sem_l1, sem_r1):
    wid = lax.axis_index("s") * NC + lax.axis_index("c")
    base = wid * BW
    iota = lax.iota(jnp.int32, 16)
    pltpu.sync_copy(rows_hbm.at[pl.ds(base, BW)], rows_v)
    pltpu.sync_copy(cols_hbm.at[pl.ds(base, BW)], cols_v)

    lbufs = (lrows0, lrows1)
    rbufs = (rrows0, rrows1)
    lsems = (sem_l0, sem_l1)
    rsems = (sem_r0, sem_r1)

    def start(s):
        bi = s & 1
        cpl = pltpu.async_copy(
            l_hbm.at[rows_v.at[pl.ds(s * BC, BC)]], lbufs[bi], lsems[bi])
        cpr = pltpu.async_copy(
            rt_hbm.at[cols_v.at[pl.ds(s * BC, BC)]], rbufs[bi], rsems[bi])
        return cpl, cpr

    cps = [start(0), None]
    for s in range(NSUB):
        if s + 1 < NSUB:
            cps[(s + 1) & 1] = start(s + 1)
        cpl, cpr = cps[s & 1]
        cpl.wait()
        cpr.wait()
        lv_ref = lbufs[s & 1]
        rv_ref = rbufs[s & 1]

        def gbody(g, _, lv_ref=lv_ref, rv_ref=rv_ref, s=s):
            res = jnp.zeros((16,), jnp.float32)
            for i in range(16):
                acc_a = jnp.zeros((16,), jnp.float32)
                acc_b = jnp.zeros((16,), jnp.float32)
                for j in range(0, KG, 2):
                    acc_a = acc_a + (lv_ref[g * 16 + i, pl.ds(j * 16, 16)]
                                     * rv_ref[g * 16 + i, pl.ds(j * 16, 16)])
                    acc_b = acc_b + (
                        lv_ref[g * 16 + i, pl.ds((j + 1) * 16, 16)]
                        * rv_ref[g * 16 + i, pl.ds((j + 1) * 16, 16)])
                acc = acc_a + acc_b
                fold_v[pl.ds(i * FS, 16)] = acc
                half = acc + fold_v[pl.ds(i * FS + 8, 16)]
                tot = half[0]
                for l in range(1, 8):
                    tot = tot + half[l]
                res = jnp.where(iota == i, tot, res)
            out_v[pl.ds(s * BC + g * 16, 16)] = res
            return 0

        lax.fori_loop(0, G, gbody, 0)

    pltpu.sync_copy(out_v, out_hbm.at[pl.ds(base, BW)])


def kernel(rows, cols, L, R):
    mesh = plsc.VectorSubcoreMesh(core_axis_name="c", subcore_axis_name="s")
    f = pl.kernel(
        _sc_body,
        out_type=jax.ShapeDtypeStruct((B,), jnp.float32),
        mesh=mesh,
        scratch_types=[
            pltpu.VMEM((BW,), jnp.int32),
            pltpu.VMEM((BW,), jnp.int32),
            pltpu.VMEM((BC, K), jnp.float32),
            pltpu.VMEM((BC, K), jnp.float32),
            pltpu.VMEM((BC, K), jnp.float32),
            pltpu.VMEM((BC, K), jnp.float32),
            pltpu.VMEM((16 * FS,), jnp.float32),
            pltpu.VMEM((BW,), jnp.float32),
            pltpu.SemaphoreType.DMA,
            pltpu.SemaphoreType.DMA,
            pltpu.SemaphoreType.DMA,
            pltpu.SemaphoreType.DMA,
        ],
    )
    rt = jnp.swapaxes(R, 0, 1)
    return f(rows.astype(jnp.int32), cols.astype(jnp.int32), L, rt)


# speedup vs baseline: 1.0631x; 1.0631x over previous
import jax
import jax.numpy as jnp
from jax import lax
from jax.experimental import pallas as pl
from jax.experimental.pallas import tpu as pltpu
from jax.experimental.pallas import tpu_sc as plsc

N = 100000
M = 100000
K = 128
B = 16384
NC = 2
NS = 16
NW = NC * NS
BW = B // NW
BC = 128
NSUB = BW // BC
G = BC // 16
KG = K // 16
FS = 24


def _sc_body(rows_hbm, cols_hbm, l_hbm, rt_hbm, out_hbm,
             rows_v, cols_v, lrows0, rrows0, lrows1, rrows1,
             fold_v, out_v, sem_l0, sem_r0, sem_l1, sem_r1):
    wid = lax.axis_index("s") * NC + lax.axis_index("c")
    base = wid * BW
    iota = lax.iota(jnp.int32, 16)
    pltpu.sync_copy(rows_hbm.at[pl.ds(base, BW)], rows_v)
    pltpu.sync_copy(cols_hbm.at[pl.ds(base, BW)], cols_v)

    lbufs = (lrows0, lrows1)
    rbufs = (rrows0, rrows1)
    lsems = (sem_l0, sem_l1)
    rsems = (sem_r0, sem_r1)

    def start(s):
        bi = s & 1
        cpl = pltpu.async_copy(
            l_hbm.at[rows_v.at[pl.ds(s * BC, BC)]], lbufs[bi], lsems[bi])
        cpr = pltpu.async_copy(
            rt_hbm.at[cols_v.at[pl.ds(s * BC, BC)]], rbufs[bi], rsems[bi])
        return cpl, cpr

    cps = [start(0), None]
    for s in range(NSUB):
        if s + 1 < NSUB:
            cps[(s + 1) & 1] = start(s + 1)
        cpl, cpr = cps[s & 1]
        cpl.wait()
        cpr.wait()
        lv_ref = lbufs[s & 1]
        rv_ref = rbufs[s & 1]

        def gbody(g, _, lv_ref=lv_ref, rv_ref=rv_ref, s=s):
            res = jnp.zeros((16,), jnp.float32)
            for i in range(16):
                acc = jnp.zeros((16,), jnp.float32)
                for j in range(KG):
                    lv = lv_ref[g * 16 + i, pl.ds(j * 16, 16)]
                    rv = rv_ref[g * 16 + i, pl.ds(j * 16, 16)]
                    acc = acc + lv * rv
                fold_v[pl.ds(i * FS, 16)] = acc
                half = acc + fold_v[pl.ds(i * FS + 8, 16)]
                fold_v[pl.ds(i * FS, 16)] = half
                quart = half + fold_v[pl.ds(i * FS + 4, 16)]
                tot = quart[0]
                for l in range(1, 4):
                    tot = tot + quart[l]
                res = jnp.where(iota == i, tot, res)
            out_v[pl.ds(s * BC + g * 16, 16)] = res
            return 0

        lax.fori_loop(0, G, gbody, 0)

    pltpu.sync_copy(out_v, out_hbm.at[pl.ds(base, BW)])


def kernel(rows, cols, L, R):
    mesh = plsc.VectorSubcoreMesh(core_axis_name="c", subcore_axis_name="s")
    f = pl.kernel(
        _sc_body,
        out_type=jax.ShapeDtypeStruct((B,), jnp.float32),
        mesh=mesh,
        scratch_types=[
            pltpu.VMEM((BW,), jnp.int32),
            pltpu.VMEM((BW,), jnp.int32),
            pltpu.VMEM((BC, K), jnp.float32),
            pltpu.VMEM((BC, K), jnp.float32),
            pltpu.VMEM((BC, K), jnp.float32),
            pltpu.VMEM((BC, K), jnp.float32),
            pltpu.VMEM((16 * FS,), jnp.float32),
            pltpu.VMEM((BW,), jnp.float32),
            pltpu.SemaphoreType.DMA,
            pltpu.SemaphoreType.DMA,
            pltpu.SemaphoreType.DMA,
            pltpu.SemaphoreType.DMA,
        ],
    )
    rt = jnp.swapaxes(R, 0, 1)
    return f(rows.astype(jnp.int32), cols.astype(jnp.int32), L, rt)


# speedup vs baseline: 1.0859x; 1.0215x over previous
import jax
import jax.numpy as jnp
from jax import lax
from jax.experimental import pallas as pl
from jax.experimental.pallas import tpu as pltpu
from jax.experimental.pallas import tpu_sc as plsc

N = 100000
M = 100000
K = 128
B = 16384
NC = 2
NS = 16
NW = NC * NS
BW = B // NW
BC = 128
NSUB = BW // BC
G = BC // 16
KG = K // 16
FS = 24


def _sc_body(rows_hbm, cols_hbm, l_hbm, rt_hbm, out_hbm,
             rows_v, cols_v, lrows0, rrows0, lrows1, rrows1,
             fold_v, out_v, sem_l0, sem_r0, sem_l1, sem_r1):
    wid = lax.axis_index("s") * NC + lax.axis_index("c")
    base = wid * BW
    iota = lax.iota(jnp.int32, 16)
    pltpu.sync_copy(rows_hbm.at[pl.ds(base, BW)], rows_v)
    pltpu.sync_copy(cols_hbm.at[pl.ds(base, BW)], cols_v)

    lbufs = (lrows0, lrows1)
    rbufs = (rrows0, rrows1)
    lsems = (sem_l0, sem_l1)
    rsems = (sem_r0, sem_r1)

    def start(s):
        bi = s & 1
        cpl = pltpu.async_copy(
            l_hbm.at[rows_v.at[pl.ds(s * BC, BC)]], lbufs[bi], lsems[bi])
        cpr = pltpu.async_copy(
            rt_hbm.at[cols_v.at[pl.ds(s * BC, BC)]], rbufs[bi], rsems[bi])
        return cpl, cpr

    cps = [start(0), None]
    for s in range(NSUB):
        if s + 1 < NSUB:
            cps[(s + 1) & 1] = start(s + 1)
        cpl, cpr = cps[s & 1]
        cpl.wait()
        cpr.wait()
        lv_ref = lbufs[s & 1]
        rv_ref = rbufs[s & 1]

        def gbody(g, _, lv_ref=lv_ref, rv_ref=rv_ref, s=s):
            res = jnp.zeros((16,), jnp.float32)
            for i in range(16):
                p = [lv_ref[g * 16 + i, pl.ds(j * 16, 16)]
                     * rv_ref[g * 16 + i, pl.ds(j * 16, 16)]
                     for j in range(KG)]
                while len(p) > 1:
                    p = [p[t] + p[t + 1] for t in range(0, len(p), 2)]
                acc = p[0]
                fold_v[pl.ds(i * FS, 16)] = acc
                half = acc + fold_v[pl.ds(i * FS + 8, 16)]
                tot = half[0]
                for l in range(1, 8):
                    tot = tot + half[l]
                res = jnp.where(iota == i, tot, res)
            out_v[pl.ds(s * BC + g * 16, 16)] = res
            return 0

        lax.fori_loop(0, G, gbody, 0)

    pltpu.sync_copy(out_v, out_hbm.at[pl.ds(base, BW)])


def kernel(rows, cols, L, R):
    mesh = plsc.VectorSubcoreMesh(core_axis_name="c", subcore_axis_name="s")
    f = pl.kernel(
        _sc_body,
        out_type=jax.ShapeDtypeStruct((B,), jnp.float32),
        mesh=mesh,
        scratch_types=[
            pltpu.VMEM((BW,), jnp.int32),
            pltpu.VMEM((BW,), jnp.int32),
            pltpu.VMEM((BC, K), jnp.float32),
            pltpu.VMEM((BC, K), jnp.float32),
            pltpu.VMEM((BC, K), jnp.float32),
            pltpu.VMEM((BC, K), jnp.float32),
            pltpu.VMEM((16 * FS,), jnp.float32),
            pltpu.VMEM((BW,), jnp.float32),
            pltpu.SemaphoreType.DMA,
            pltpu.SemaphoreType.DMA,
            pltpu.SemaphoreType.DMA,
            pltpu.SemaphoreType.DMA,
        ],
    )
    rt = jnp.swapaxes(R, 0, 1)
    return f(rows.astype(jnp.int32), cols.astype(jnp.int32), L, rt)


# speedup vs baseline: 1.1321x; 1.0426x over previous
import jax
import jax.numpy as jnp
from jax import lax
from jax.experimental import pallas as pl
from jax.experimental.pallas import tpu as pltpu
from jax.experimental.pallas import tpu_sc as plsc

N = 100000
M = 100000
K = 128
B = 16384
NC = 2
NS = 16
NW = NC * NS
BW = B // NW
BC = 128
NSUB = BW // BC
G = BC // 16
KG = K // 16
FS = 24


def _sc_body(rows_hbm, cols_hbm, l_hbm, rt_hbm, out_hbm,
             rows_v, cols_v, lrows0, rrows0, lrows1, rrows1,
             fold_v, out_v, sem_l0, sem_r0, sem_l1, sem_r1):
    wid = lax.axis_index("s") * NC + lax.axis_index("c")
    base = wid * BW
    iota = lax.iota(jnp.int32, 16)
    pltpu.sync_copy(rows_hbm.at[pl.ds(base, BW)], rows_v)
    pltpu.sync_copy(cols_hbm.at[pl.ds(base, BW)], cols_v)

    lbufs = (lrows0, lrows1)
    rbufs = (rrows0, rrows1)
    lsems = (sem_l0, sem_l1)
    rsems = (sem_r0, sem_r1)

    def start(s):
        bi = s & 1
        cpl = pltpu.async_copy(
            l_hbm.at[rows_v.at[pl.ds(s * BC, BC)]], lbufs[bi], lsems[bi])
        cpr = pltpu.async_copy(
            rt_hbm.at[cols_v.at[pl.ds(s * BC, BC)]], rbufs[bi], rsems[bi])
        return cpl, cpr

    cps = [start(0), None]
    for s in range(NSUB):
        if s + 1 < NSUB:
            cps[(s + 1) & 1] = start(s + 1)
        cpl, cpr = cps[s & 1]
        cpl.wait()
        cpr.wait()
        lv_ref = lbufs[s & 1]
        rv_ref = rbufs[s & 1]

        def gbody(g, _, lv_ref=lv_ref, rv_ref=rv_ref, s=s):
            res = jnp.zeros((16,), jnp.float32)
            for i in range(16):
                acc = jnp.zeros((16,), jnp.float32)
                for j in range(KG):
                    lv = lv_ref[g * 16 + i, pl.ds(j * 16, 16)]
                    rv = rv_ref[g * 16 + i, pl.ds(j * 16, 16)]
                    acc = acc + lv * rv
                fold_v[pl.ds(i * FS, 16)] = acc
                half = acc + fold_v[pl.ds(i * FS + 8, 16)]
                tot = half[0]
                for l in range(1, 8):
                    tot = tot + half[l]
                res = jnp.where(iota == i, tot, res)
            out_v[pl.ds(s * BC + g * 16, 16)] = res
            return 0

        lax.fori_loop(0, G, gbody, 0)

    pltpu.sync_copy(out_v, out_hbm.at[pl.ds(base, BW)])


def kernel(rows, cols, L, R):
    mesh = plsc.VectorSubcoreMesh(core_axis_name="c", subcore_axis_name="s")
    f = pl.kernel(
        _sc_body,
        out_type=jax.ShapeDtypeStruct((B,), jnp.float32),
        mesh=mesh,
        scratch_types=[
            pltpu.VMEM((BW,), jnp.int32),
            pltpu.VMEM((BW,), jnp.int32),
            pltpu.VMEM((BC, K), jnp.float32),
            pltpu.VMEM((BC, K), jnp.float32),
            pltpu.VMEM((BC, K), jnp.float32),
            pltpu.VMEM((BC, K), jnp.float32),
            pltpu.VMEM((16 * FS,), jnp.float32),
            pltpu.VMEM((BW,), jnp.float32),
            pltpu.SemaphoreType.DMA,
            pltpu.SemaphoreType.DMA,
            pltpu.SemaphoreType.DMA,
            pltpu.SemaphoreType.DMA,
        ],
    )
    rt = jnp.swapaxes(R, 0, 1)
    return f(rows.astype(jnp.int32), cols.astype(jnp.int32), L, rt)
